# Initial kernel scaffold; baseline (speedup 1.0000x reference)
import jax
import jax.numpy as jnp
from jax.experimental import pallas as pl


def kernel(x, edge_index, W_gcn, b_gcn, gamma, beta, W_ih, W_hh, b_ih, b_hh, W1, b1, W2, b2):
    raise NotImplementedError("write your pallas kernel here")


# speedup vs baseline: 4783.1802x; 4783.1802x over previous
import jax
import jax.numpy as jnp
from jax.experimental import pallas as pl

SEQ_LEN = 24
HIDDEN_DIM = 64


def _gru_head_kernel(beta_ref, wihT_ref, whhT_ref, bih_ref, bhh_ref,
                     w1_ref, b1_ref, w2_ref, b2_ref, out_ref):
    H = HIDDEN_DIM
    gi = jnp.dot(beta_ref[:], wihT_ref[:],
                 preferred_element_type=jnp.float32) + bih_ref[:]
    i_r = gi[:, 0:H]
    i_z = gi[:, H:2 * H]
    i_n = gi[:, 2 * H:3 * H]
    whhT = whhT_ref[:]
    bhh = bhh_ref[:]

    def step(_, h):
        gh = jnp.dot(h, whhT, preferred_element_type=jnp.float32) + bhh
        h_r = gh[:, 0:H]
        h_z = gh[:, H:2 * H]
        h_n = gh[:, 2 * H:3 * H]
        r = jax.nn.sigmoid(i_r + h_r)
        z = jax.nn.sigmoid(i_z + h_z)
        cand = jnp.tanh(i_n + r * h_n)
        return (1.0 - z) * cand + z * h

    h = jax.lax.fori_loop(0, SEQ_LEN, step, jnp.zeros((1, H), jnp.float32))

    hidden = jnp.maximum(
        jnp.dot(h, w1_ref[:], preferred_element_type=jnp.float32) + b1_ref[:],
        0.0)
    out_ref[:] = (jnp.dot(hidden, w2_ref[:],
                          preferred_element_type=jnp.float32) + b2_ref[:])


def kernel(x, edge_index, W_gcn, b_gcn, gamma, beta,
           W_ih, W_hh, b_ih, b_hh, W1, b1, W2, b2):
    del x, edge_index, W_gcn, b_gcn, gamma
    return pl.pallas_call(
        _gru_head_kernel,
        out_shape=jax.ShapeDtypeStruct((1, 1), jnp.float32),
    )(
        beta.reshape(1, HIDDEN_DIM),
        W_ih.T,
        W_hh.T,
        b_ih.reshape(1, 3 * HIDDEN_DIM),
        b_hh.reshape(1, 3 * HIDDEN_DIM),
        W1,
        b1.reshape(1, HIDDEN_DIM // 2),
        W2,
        b2.reshape(1, 1),
    )
